# Initial kernel scaffold; baseline (speedup 1.0000x reference)
import jax
import jax.numpy as jnp
from jax.experimental import pallas as pl


def kernel(x, edge_index, W0, b0, g0, be0, W1, b1, g1, be1, W2, b2, g2, be2, Wc, bc):
    raise NotImplementedError("write your pallas kernel here")


# speedup vs baseline: 5.8372x; 5.8372x over previous
import functools

import jax
import jax.numpy as jnp
from jax import lax
from jax.experimental import pallas as pl
from jax.experimental.pallas import tpu as pltpu
from jax.experimental.pallas import tpu_sc as plsc

NODES = 10000
DIM = 128
CLS = 40
NC = 2
NS = 16
LANES = 16
NW = NC * NS
NP = 10240
RPT = NP // NS
K = 128
BR = 512

f32 = jnp.float32


def _fill_own_rows(idx2, s):
    iota16 = jax.lax.iota(jnp.int32, LANES)
    base = s * RPT
    for r in range(RPT // K):
        for cc in range(K // LANES):
            idx2[r, pl.ds(cc * LANES, LANES)] = (
                base + r * K + cc * LANES + iota16)


def _deg_body(dst_hbm, out_hbm, dstv, buf0, stage16, idx2, acc_sh):
    ch = dstv.shape[0]
    c = lax.axis_index("c")
    s = lax.axis_index("s")
    zeros16 = jnp.zeros((LANES,), f32)
    ones16 = jnp.ones((LANES,), f32)
    pltpu.sync_copy(dst_hbm.at[c, s], dstv)
    _fill_own_rows(idx2, s)

    def fill_zero(i, carry):
        for t in range(DIM // LANES):
            buf0[i, pl.ds(t * LANES, LANES)] = zeros16
        return carry

    lax.fori_loop(0, K, fill_zero, 0)
    for r in range(RPT // K):
        pltpu.sync_copy(buf0, acc_sh.at[idx2.at[r]])
    plsc.subcore_barrier()

    def fill_ones(i, carry):
        for t in range(DIM // LANES):
            buf0[i, pl.ds(t * LANES, LANES)] = ones16
        return carry

    lax.fori_loop(0, K, fill_ones, 0)
    for j in range(ch):
        pltpu.sync_copy(buf0, acc_sh.at[dstv.at[j]], add=True)
    plsc.subcore_barrier()
    for r in range(RPT // K):
        pltpu.sync_copy(acc_sh.at[idx2.at[r]], buf0)

        def extract(i, carry):
            stage16[i, :] = buf0[i, 0:LANES]
            return carry

        lax.fori_loop(0, K, extract, 0)
        pltpu.sync_copy(stage16, out_hbm.at[c, pl.ds(s * RPT + r * K, K)])


def _make_deg_kernel(ch):
    mesh = plsc.VectorSubcoreMesh(core_axis_name="c", subcore_axis_name="s",
                                  num_cores=NC, num_subcores=NS)
    return pl.kernel(
        _deg_body,
        out_type=jax.ShapeDtypeStruct((NC, NP, LANES), f32),
        mesh=mesh,
        scratch_types=[
            pltpu.VMEM((ch, K), jnp.int32),
            pltpu.VMEM((K, DIM), f32),
            pltpu.VMEM((K, LANES), f32),
            pltpu.VMEM((RPT // K, K), jnp.int32),
            pltpu.VMEM_SHARED((NP, DIM), f32),
        ],
    )


GRP = 16


def _agg_body(xs_hbm, src_hbm, dst_hbm, out_hbm,
              srcv, dstv, buf0, buf1, idx2, acc_sh, sem0, sem1):
    ng = src_hbm.shape[2] // GRP
    c = lax.axis_index("c")
    s = lax.axis_index("s")
    zeros16 = jnp.zeros((LANES,), f32)
    _fill_own_rows(idx2, s)

    def fill_zero(i, carry):
        for t in range(DIM // LANES):
            buf0[i, pl.ds(t * LANES, LANES)] = zeros16
        return carry

    lax.fori_loop(0, K, fill_zero, 0)
    for r in range(RPT // K):
        pltpu.sync_copy(buf0, acc_sh.at[idx2.at[r]])
    plsc.subcore_barrier()

    def group(gi, carry):
        pltpu.sync_copy(src_hbm.at[c, s, pl.ds(gi * GRP, GRP)], srcv)
        pltpu.sync_copy(dst_hbm.at[c, s, pl.ds(gi * GRP, GRP)], dstv)
        for jj in range(0, GRP, 2):
            cp0 = pltpu.async_copy(xs_hbm.at[srcv.at[jj]], buf0, sem0)
            cp1 = pltpu.async_copy(xs_hbm.at[srcv.at[jj + 1]], buf1, sem1)
            cp0.wait()
            pltpu.sync_copy(buf0, acc_sh.at[dstv.at[jj]], add=True)
            cp1.wait()
            pltpu.sync_copy(buf1, acc_sh.at[dstv.at[jj + 1]], add=True)
        return carry

    lax.fori_loop(0, ng, group, 0)
    plsc.subcore_barrier()
    for r in range(RPT // K):
        pltpu.sync_copy(acc_sh.at[idx2.at[r]], buf0)
        pltpu.sync_copy(buf0, out_hbm.at[c, pl.ds(s * RPT + r * K, K)])


def _make_agg_kernel(ch):
    mesh = plsc.VectorSubcoreMesh(core_axis_name="c", subcore_axis_name="s",
                                  num_cores=NC, num_subcores=NS)
    return pl.kernel(
        _agg_body,
        out_type=jax.ShapeDtypeStruct((NC, NP, DIM), f32),
        mesh=mesh,
        scratch_types=[
            pltpu.VMEM((GRP, K), jnp.int32),
            pltpu.VMEM((GRP, K), jnp.int32),
            pltpu.VMEM((K, DIM), f32),
            pltpu.VMEM((K, DIM), f32),
            pltpu.VMEM((RPT // K, K), jnp.int32),
            pltpu.VMEM_SHARED((NP, DIM), f32),
            pltpu.SemaphoreType.DMA,
            pltpu.SemaphoreType.DMA,
        ],
    )


def _dinv_body(degp_ref, o_ref):
    d = degp_ref[0] + degp_ref[1]
    dt = jnp.sum(d, axis=-1, keepdims=True) * (1.0 / LANES) + 1.0
    o_ref[...] = lax.rsqrt(dt)


def _dinv_call(degp):
    grid = (NP // BR,)
    return pl.pallas_call(
        _dinv_body,
        grid=grid,
        in_specs=[pl.BlockSpec((NC, BR, LANES), lambda i: (0, i, 0))],
        out_specs=pl.BlockSpec((BR, 1), lambda i: (i, 0)),
        out_shape=jax.ShapeDtypeStruct((NP, 1), f32),
    )(degp)


def _mm_scale_body(x_ref, w_ref, dinv_ref, o_ref):
    xw = jnp.dot(x_ref[...], w_ref[...], preferred_element_type=f32)
    o_ref[...] = xw * dinv_ref[...]


def _mm_scale_call(x, w, dinv):
    grid = (NP // BR,)
    return pl.pallas_call(
        _mm_scale_body,
        grid=grid,
        in_specs=[
            pl.BlockSpec((BR, DIM), lambda i: (i, 0)),
            pl.BlockSpec((DIM, DIM), lambda i: (0, 0)),
            pl.BlockSpec((BR, 1), lambda i: (i, 0)),
        ],
        out_specs=pl.BlockSpec((BR, DIM), lambda i: (i, 0)),
        out_shape=jax.ShapeDtypeStruct((NP, DIM), f32),
    )(x, w, dinv)


def _ln_body(accp_ref, xs_ref, dinv_ref, x_ref, b_ref, g_ref, be_ref, o_ref):
    acc = accp_ref[0] + accp_ref[1]
    agg = (acc + xs_ref[...]) * dinv_ref[...] + b_ref[...]
    mu = jnp.mean(agg, axis=-1, keepdims=True)
    cen = agg - mu
    var = jnp.mean(cen * cen, axis=-1, keepdims=True)
    ln = cen * lax.rsqrt(var + 1e-5) * g_ref[...] + be_ref[...]
    o_ref[...] = x_ref[...] + jnp.maximum(ln, 0.0)


def _ln_call(accp, xs, dinv, x, b, g, be):
    grid = (NP // BR,)
    return pl.pallas_call(
        _ln_body,
        grid=grid,
        in_specs=[
            pl.BlockSpec((NC, BR, DIM), lambda i: (0, i, 0)),
            pl.BlockSpec((BR, DIM), lambda i: (i, 0)),
            pl.BlockSpec((BR, 1), lambda i: (i, 0)),
            pl.BlockSpec((BR, DIM), lambda i: (i, 0)),
            pl.BlockSpec((1, DIM), lambda i: (0, 0)),
            pl.BlockSpec((1, DIM), lambda i: (0, 0)),
            pl.BlockSpec((1, DIM), lambda i: (0, 0)),
        ],
        out_specs=pl.BlockSpec((BR, DIM), lambda i: (i, 0)),
        out_shape=jax.ShapeDtypeStruct((NP, DIM), f32),
    )(accp, xs, dinv, x, b, g, be)


def _cls_body(x_ref, w_ref, b_ref, o_ref):
    o_ref[...] = jnp.dot(x_ref[...], w_ref[...],
                         preferred_element_type=f32) + b_ref[...]


def _cls_call(x, wc_pad, bc_pad):
    grid = (NP // BR,)
    return pl.pallas_call(
        _cls_body,
        grid=grid,
        in_specs=[
            pl.BlockSpec((BR, DIM), lambda i: (i, 0)),
            pl.BlockSpec((DIM, DIM), lambda i: (0, 0)),
            pl.BlockSpec((1, DIM), lambda i: (0, 0)),
        ],
        out_specs=pl.BlockSpec((BR, DIM), lambda i: (i, 0)),
        out_shape=jax.ShapeDtypeStruct((NP, DIM), f32),
    )(x, wc_pad, bc_pad)


@jax.jit
def _run(x, edge_index, W0, b0, g0, be0, W1, b1, g1, be1, W2, b2, g2, be2,
         Wc, bc):
    e = edge_index.shape[1]
    ch = -(-e // (NW * K))
    ch = -(-ch // GRP) * GRP
    ep = NW * ch * K

    xp = jnp.pad(x, ((0, NP - NODES), (0, 0)))
    pad = jnp.full((ep - e,), NODES, jnp.int32)
    src = jnp.concatenate([edge_index[0], pad]).reshape(NC, NS, ch, K)
    dst = jnp.concatenate([edge_index[1], pad]).reshape(NC, NS, ch, K)

    degp = _make_deg_kernel(ch)(dst)
    dinv = _dinv_call(degp)

    edge_agg = _make_agg_kernel(ch)
    xcur = xp
    for (W, b, g, be) in ((W0, b0, g0, be0), (W1, b1, g1, be1),
                          (W2, b2, g2, be2)):
        xs = _mm_scale_call(xcur, W, dinv)
        accp = edge_agg(xs, src, dst)
        xcur = _ln_call(accp, xs, dinv, xcur,
                        b.reshape(1, DIM), g.reshape(1, DIM),
                        be.reshape(1, DIM))

    wc_pad = jnp.pad(Wc, ((0, 0), (0, DIM - CLS)))
    bc_pad = jnp.pad(bc, ((0, DIM - CLS),)).reshape(1, DIM)
    out = _cls_call(xcur, wc_pad, bc_pad)
    return out[:NODES, :CLS]


def kernel(x, edge_index, W0, b0, g0, be0, W1, b1, g1, be1, W2, b2, g2, be2,
           Wc, bc):
    return _run(x, edge_index, W0, b0, g0, be0, W1, b1, g1, be1,
                W2, b2, g2, be2, Wc, bc)


# speedup vs baseline: 6.2082x; 1.0636x over previous
import functools

import jax
import jax.numpy as jnp
from jax import lax
from jax.experimental import pallas as pl
from jax.experimental.pallas import tpu as pltpu
from jax.experimental.pallas import tpu_sc as plsc

NODES = 10000
DIM = 128
CLS = 40
NC = 2
NS = 16
LANES = 16
NW = NC * NS
NP = 10240
RPT = NP // NS
K = 128
BR = 512

f32 = jnp.float32


def _fill_own_rows(idx2, s, kchunk):
    iota16 = jax.lax.iota(jnp.int32, LANES)
    base = s * RPT
    for r in range(RPT // kchunk):
        for cc in range(kchunk // LANES):
            idx2[r, pl.ds(cc * LANES, LANES)] = (
                base + r * kchunk + cc * LANES + iota16)


def _deg_body(dst_hbm, out_hbm, dstv, buf0, stage16, idx2, acc_sh):
    ch = dstv.shape[0]
    c = lax.axis_index("c")
    s = lax.axis_index("s")
    zeros16 = jnp.zeros((LANES,), f32)
    ones16 = jnp.ones((LANES,), f32)
    pltpu.sync_copy(dst_hbm.at[c, s], dstv)
    _fill_own_rows(idx2, s, K)

    def fill_zero(i, carry):
        for t in range(DIM // LANES):
            buf0[i, pl.ds(t * LANES, LANES)] = zeros16
        return carry

    lax.fori_loop(0, K, fill_zero, 0)
    for r in range(RPT // K):
        pltpu.sync_copy(buf0, acc_sh.at[idx2.at[r]])
    plsc.subcore_barrier()

    def fill_ones(i, carry):
        for t in range(DIM // LANES):
            buf0[i, pl.ds(t * LANES, LANES)] = ones16
        return carry

    lax.fori_loop(0, K, fill_ones, 0)
    for j in range(ch):
        pltpu.sync_copy(buf0, acc_sh.at[dstv.at[j]], add=True)
    plsc.subcore_barrier()
    for r in range(RPT // K):
        pltpu.sync_copy(acc_sh.at[idx2.at[r]], buf0)

        def extract(i, carry):
            stage16[i, :] = buf0[i, 0:LANES]
            return carry

        lax.fori_loop(0, K, extract, 0)
        pltpu.sync_copy(stage16, out_hbm.at[c, pl.ds(s * RPT + r * K, K)])


def _make_deg_kernel(ch):
    mesh = plsc.VectorSubcoreMesh(core_axis_name="c", subcore_axis_name="s",
                                  num_cores=NC, num_subcores=NS)
    return pl.kernel(
        _deg_body,
        out_type=jax.ShapeDtypeStruct((NC, NP, LANES), f32),
        mesh=mesh,
        scratch_types=[
            pltpu.VMEM((ch, K), jnp.int32),
            pltpu.VMEM((K, DIM), f32),
            pltpu.VMEM((K, LANES), f32),
            pltpu.VMEM((RPT // K, K), jnp.int32),
            pltpu.VMEM_SHARED((NP, DIM), f32),
        ],
    )


GRP = 16
KE = 80
NBUF = 3


def _agg_body(xs_hbm, src_hbm, dst_hbm, out_hbm,
              srcv, dstv, bufs, idx2, acc_sh, gsems, ssems):
    ng = src_hbm.shape[2] // GRP
    c = lax.axis_index("c")
    s = lax.axis_index("s")
    zeros16 = jnp.zeros((LANES,), f32)
    _fill_own_rows(idx2, s, KE)

    def fill_zero(i, carry):
        for t in range(DIM // LANES):
            bufs[0][i, pl.ds(t * LANES, LANES)] = zeros16
        return carry

    lax.fori_loop(0, KE, fill_zero, 0)
    for r in range(RPT // KE):
        pltpu.sync_copy(bufs[0], acc_sh.at[idx2.at[r]])
    plsc.subcore_barrier()

    def group(gi, carry):
        pltpu.sync_copy(src_hbm.at[c, s, pl.ds(gi * GRP, GRP)], srcv)
        pltpu.sync_copy(dst_hbm.at[c, s, pl.ds(gi * GRP, GRP)], dstv)
        g = [None] * NBUF
        sct = [None] * NBUF
        for jj in range(min(2, GRP)):
            g[jj % NBUF] = pltpu.async_copy(
                xs_hbm.at[srcv.at[jj]], bufs[jj % NBUF], gsems[jj % NBUF])
        for jj in range(GRP):
            b = jj % NBUF
            g[b].wait()
            sct[b] = pltpu.async_copy(bufs[b], acc_sh.at[dstv.at[jj]],
                                      ssems[b], add=True)
            nxt = jj + 2
            if nxt < GRP:
                nb = nxt % NBUF
                if sct[nb] is not None:
                    sct[nb].wait()
                    sct[nb] = None
                g[nb] = pltpu.async_copy(xs_hbm.at[srcv.at[nxt]], bufs[nb],
                                         gsems[nb])
        for b in range(NBUF):
            if sct[b] is not None:
                sct[b].wait()
        return carry

    lax.fori_loop(0, ng, group, 0)
    plsc.subcore_barrier()
    for r in range(RPT // KE):
        b = r % NBUF
        pltpu.sync_copy(acc_sh.at[idx2.at[r]], bufs[b])
        pltpu.sync_copy(bufs[b], out_hbm.at[c, pl.ds(s * RPT + r * KE, KE)])


def _make_agg_kernel(ch):
    mesh = plsc.VectorSubcoreMesh(core_axis_name="c", subcore_axis_name="s",
                                  num_cores=NC, num_subcores=NS)
    return pl.kernel(
        _agg_body,
        out_type=jax.ShapeDtypeStruct((NC, NP, DIM), f32),
        mesh=mesh,
        scratch_types=[
            pltpu.VMEM((GRP, KE), jnp.int32),
            pltpu.VMEM((GRP, KE), jnp.int32),
            [pltpu.VMEM((KE, DIM), f32) for _ in range(NBUF)],
            pltpu.VMEM((RPT // KE, KE), jnp.int32),
            pltpu.VMEM_SHARED((NP, DIM), f32),
            [pltpu.SemaphoreType.DMA for _ in range(NBUF)],
            [pltpu.SemaphoreType.DMA for _ in range(NBUF)],
        ],
    )


def _dinv_body(degp_ref, o_ref):
    d = degp_ref[0] + degp_ref[1]
    dt = jnp.sum(d, axis=-1, keepdims=True) * (1.0 / LANES) + 1.0
    o_ref[...] = lax.rsqrt(dt)


def _dinv_call(degp):
    grid = (NP // BR,)
    return pl.pallas_call(
        _dinv_body,
        grid=grid,
        in_specs=[pl.BlockSpec((NC, BR, LANES), lambda i: (0, i, 0))],
        out_specs=pl.BlockSpec((BR, 1), lambda i: (i, 0)),
        out_shape=jax.ShapeDtypeStruct((NP, 1), f32),
    )(degp)


def _mm_scale_body(x_ref, w_ref, dinv_ref, o_ref):
    xw = jnp.dot(x_ref[...], w_ref[...], preferred_element_type=f32)
    o_ref[...] = xw * dinv_ref[...]


def _mm_scale_call(x, w, dinv):
    grid = (NP // BR,)
    return pl.pallas_call(
        _mm_scale_body,
        grid=grid,
        in_specs=[
            pl.BlockSpec((BR, DIM), lambda i: (i, 0)),
            pl.BlockSpec((DIM, DIM), lambda i: (0, 0)),
            pl.BlockSpec((BR, 1), lambda i: (i, 0)),
        ],
        out_specs=pl.BlockSpec((BR, DIM), lambda i: (i, 0)),
        out_shape=jax.ShapeDtypeStruct((NP, DIM), f32),
    )(x, w, dinv)


def _ln_body(accp_ref, xs_ref, dinv_ref, x_ref, b_ref, g_ref, be_ref, o_ref):
    acc = accp_ref[0] + accp_ref[1]
    agg = (acc + xs_ref[...]) * dinv_ref[...] + b_ref[...]
    mu = jnp.mean(agg, axis=-1, keepdims=True)
    cen = agg - mu
    var = jnp.mean(cen * cen, axis=-1, keepdims=True)
    ln = cen * lax.rsqrt(var + 1e-5) * g_ref[...] + be_ref[...]
    o_ref[...] = x_ref[...] + jnp.maximum(ln, 0.0)


def _ln_call(accp, xs, dinv, x, b, g, be):
    grid = (NP // BR,)
    return pl.pallas_call(
        _ln_body,
        grid=grid,
        in_specs=[
            pl.BlockSpec((NC, BR, DIM), lambda i: (0, i, 0)),
            pl.BlockSpec((BR, DIM), lambda i: (i, 0)),
            pl.BlockSpec((BR, 1), lambda i: (i, 0)),
            pl.BlockSpec((BR, DIM), lambda i: (i, 0)),
            pl.BlockSpec((1, DIM), lambda i: (0, 0)),
            pl.BlockSpec((1, DIM), lambda i: (0, 0)),
            pl.BlockSpec((1, DIM), lambda i: (0, 0)),
        ],
        out_specs=pl.BlockSpec((BR, DIM), lambda i: (i, 0)),
        out_shape=jax.ShapeDtypeStruct((NP, DIM), f32),
    )(accp, xs, dinv, x, b, g, be)


def _cls_body(x_ref, w_ref, b_ref, o_ref):
    o_ref[...] = jnp.dot(x_ref[...], w_ref[...],
                         preferred_element_type=f32) + b_ref[...]


def _cls_call(x, wc_pad, bc_pad):
    grid = (NP // BR,)
    return pl.pallas_call(
        _cls_body,
        grid=grid,
        in_specs=[
            pl.BlockSpec((BR, DIM), lambda i: (i, 0)),
            pl.BlockSpec((DIM, DIM), lambda i: (0, 0)),
            pl.BlockSpec((1, DIM), lambda i: (0, 0)),
        ],
        out_specs=pl.BlockSpec((BR, DIM), lambda i: (i, 0)),
        out_shape=jax.ShapeDtypeStruct((NP, DIM), f32),
    )(x, wc_pad, bc_pad)


@jax.jit
def _run(x, edge_index, W0, b0, g0, be0, W1, b1, g1, be1, W2, b2, g2, be2,
         Wc, bc):
    e = edge_index.shape[1]
    chd = -(-e // (NW * K))
    epd = NW * chd * K
    cha = -(-e // (NW * KE))
    cha = -(-cha // GRP) * GRP
    epa = NW * cha * KE

    xp = jnp.pad(x, ((0, NP - NODES), (0, 0)))
    padd = jnp.full((epd - e,), NODES, jnp.int32)
    dstd = jnp.concatenate([edge_index[1], padd]).reshape(NC, NS, chd, K)
    pada = jnp.full((epa - e,), NODES, jnp.int32)
    src = jnp.concatenate([edge_index[0], pada]).reshape(NC, NS, cha, KE)
    dst = jnp.concatenate([edge_index[1], pada]).reshape(NC, NS, cha, KE)

    degp = _make_deg_kernel(chd)(dstd)
    dinv = _dinv_call(degp)

    edge_agg = _make_agg_kernel(cha)
    xcur = xp
    for (W, b, g, be) in ((W0, b0, g0, be0), (W1, b1, g1, be1),
                          (W2, b2, g2, be2)):
        xs = _mm_scale_call(xcur, W, dinv)
        accp = edge_agg(xs, src, dst)
        xcur = _ln_call(accp, xs, dinv, xcur,
                        b.reshape(1, DIM), g.reshape(1, DIM),
                        be.reshape(1, DIM))

    wc_pad = jnp.pad(Wc, ((0, 0), (0, DIM - CLS)))
    bc_pad = jnp.pad(bc, ((0, DIM - CLS),)).reshape(1, DIM)
    out = _cls_call(xcur, wc_pad, bc_pad)
    return out[:NODES, :CLS]


def kernel(x, edge_index, W0, b0, g0, be0, W1, b1, g1, be1, W2, b2, g2, be2,
           Wc, bc):
    return _run(x, edge_index, W0, b0, g0, be0, W1, b1, g1, be1,
                W2, b2, g2, be2, Wc, bc)
